# baseline (speedup 1.0000x reference)
import functools

import jax
import jax.numpy as jnp
from jax import lax
from jax.experimental import pallas as pl
from jax.experimental.pallas import tpu as pltpu
from jax.experimental.pallas import tpu_sc as plsc

D = 256
LANES = 16
C = 32
N_WORKERS = 32
RPT = 320
QUAD = 4 * C
N_LAYERS = 3


def _sc_spmm(table, src, dstl, valb, meta, *, e_cap):
    n_total = table.shape[0]
    mesh = plsc.VectorSubcoreMesh(core_axis_name="c", subcore_axis_name="s")

    @functools.partial(
        pl.kernel,
        out_type=jax.ShapeDtypeStruct((n_total, D), jnp.float32),
        mesh=mesh,
        scratch_types=[
            pltpu.VMEM((RPT, D), jnp.float32),
            pltpu.VMEM((C, D), jnp.float32),
            pltpu.VMEM((C, D), jnp.float32),
            [pltpu.VMEM((C, LANES), jnp.float32) for _ in range(4)],
            [pltpu.VMEM((C,), jnp.int32) for _ in range(4)],
            [pltpu.VMEM((C,), jnp.int32) for _ in range(4)],
            pltpu.VMEM((2 * LANES,), jnp.int32),
            pltpu.SemaphoreType.DMA,
            pltpu.SemaphoreType.DMA,
            [pltpu.SemaphoreType.DMA for _ in range(4)],
            [pltpu.SemaphoreType.DMA for _ in range(4)],
            [pltpu.SemaphoreType.DMA for _ in range(4)],
            pltpu.SemaphoreType.DMA,
        ],
    )
    def spmm_kernel(table_h, src_h, dstl_h, valb_h, meta_h, out_h,
                    acc, g0, g1, vr, sr, dr, metav,
                    gsem0, gsem1, vsem, ssem, dsem, msem):
        core = lax.axis_index("c")
        sub = lax.axis_index("s")
        wid = sub * 2 + core
        gbufs = (g0, g1)
        gsems = (gsem0, gsem1)

        pltpu.async_copy(meta_h.at[pl.ds(wid * LANES, LANES)],
                         metav.at[pl.ds(0, LANES)], msem)
        pltpu.async_copy(meta_h.at[pl.ds((N_WORKERS + wid) * LANES, LANES)],
                         metav.at[pl.ds(LANES, LANES)], msem)

        zero = jnp.zeros((LANES,), jnp.float32)

        def zrow(j, carry):
            for k in range(D // LANES):
                acc[j, pl.ds(k * LANES, LANES)] = zero
            return carry

        lax.fori_loop(0, RPT, zrow, 0)

        pltpu.make_async_copy(meta_h.at[pl.ds(wid * LANES, LANES)],
                              metav.at[pl.ds(0, LANES)], msem).wait()
        pltpu.make_async_copy(meta_h.at[pl.ds((N_WORKERS + wid) * LANES, LANES)],
                              metav.at[pl.ds(LANES, LANES)], msem).wait()
        nch = metav[pl.ds(0, LANES)][0]
        pst = pl.multiple_of(metav[pl.ds(LANES, LANES)][0], QUAD)

        def idx_start(ci, q):
            off = pst + ci * C
            pltpu.async_copy(src_h.at[pl.ds(off, C)], sr[q], ssem[q])
            pltpu.async_copy(dstl_h.at[pl.ds(off, C)], dr[q], dsem[q])
            pltpu.async_copy(valb_h.at[pl.ds(off, C)], vr[q], vsem[q])

        def idx_wait_src(ci, q):
            off = pst + ci * C
            pltpu.make_async_copy(src_h.at[pl.ds(off, C)], sr[q], ssem[q]).wait()

        def idx_wait_rest(ci, q):
            off = pst + ci * C
            pltpu.make_async_copy(dstl_h.at[pl.ds(off, C)], dr[q], dsem[q]).wait()
            pltpu.make_async_copy(valb_h.at[pl.ds(off, C)], vr[q], vsem[q]).wait()

        def rows_start(q, b):
            pltpu.async_copy(table_h.at[sr[q]], gbufs[b], gsems[b])

        def rows_wait(q, b):
            pltpu.make_async_copy(table_h.at[sr[q]], gbufs[b], gsems[b]).wait()

        @pl.when(nch > 0)
        def _():
            for q in range(4):
                idx_start(q, q)
            for q in range(2):
                idx_wait_src(q, q)
                rows_start(q, q)

        def chunk_quad(g, carry):
            ci0 = g * 4
            for u in range(4):
                ci = ci0 + u
                b = u % 2
                gb = gbufs[b]
                rows_wait(u, b)
                idx_wait_rest(ci, u)

                def group(jg, carry2):
                    j0 = jg * LANES
                    dvec = dr[u][pl.ds(j0, LANES)]
                    for jj in range(LANES):
                        r = dvec[jj]
                        vv = vr[u][j0 + jj]
                        for k in range(D // LANES):
                            sl = pl.ds(k * LANES, LANES)
                            acc[r, sl] = acc[r, sl] + gb[j0 + jj, sl] * vv
                    return carry2

                lax.fori_loop(0, C // LANES, group, 0)

                @pl.when(ci + 4 < nch)
                def _():
                    idx_start(ci + 4, u)

                @pl.when(ci + 2 < nch)
                def _():
                    idx_wait_src(ci + 2, (u + 2) % 4)
                    rows_start((u + 2) % 4, b)
            return carry

        lax.fori_loop(0, nch // 4, chunk_quad, 0)

        rbase = wid * RPT

        @pl.when(wid < N_WORKERS - 1)
        def _():
            pltpu.sync_copy(acc.at[pl.ds(0, RPT)], out_h.at[pl.ds(rbase, RPT)])

        @pl.when(wid == N_WORKERS - 1)
        def _():
            last = n_total - (N_WORKERS - 1) * RPT
            pltpu.sync_copy(acc.at[pl.ds(0, last)], out_h.at[pl.ds(rbase, last)])

    return spmm_kernel(table, src, dstl, valb, meta)


def _tc_finish(x0, l1, l2, l3, uHyper, iHyper):
    n_total = x0.shape[0]
    bl = 1000
    nb = n_total // bl
    half_blocks = nb // 2

    def body(x0r, l1r, l2r, l3r, uhr, ihr, emb_r, hyp_r, gscr):
        i = pl.program_id(0)

        @pl.when(i == 0)
        def _():
            gscr[0] = lax.dot_general(uhr[...], uhr[...],
                                      (((0,), (0,)), ((), ())),
                                      preferred_element_type=jnp.float32)
            gscr[1] = lax.dot_general(ihr[...], ihr[...],
                                      (((0,), (0,)), ((), ())),
                                      preferred_element_type=jnp.float32)

        e = x0r[...] + l1r[...] + l2r[...] + l3r[...]
        emb_r[...] = e
        g = jnp.where(i < half_blocks, gscr[0], gscr[1])
        hyp_r[...] = jnp.dot(e, g, preferred_element_type=jnp.float32)

    blk = pl.BlockSpec((bl, D), lambda i: (i, 0))
    full = pl.BlockSpec(uHyper.shape, lambda i: (0, 0))
    return pl.pallas_call(
        body,
        grid=(nb,),
        in_specs=[blk, blk, blk, blk, full, full],
        out_specs=[blk, blk],
        out_shape=[jax.ShapeDtypeStruct((n_total, D), jnp.float32),
                   jax.ShapeDtypeStruct((n_total, D), jnp.float32)],
        scratch_shapes=[pltpu.VMEM((2, D, D), jnp.float32)],
    )(x0, l1, l2, l3, uHyper, iHyper)


def kernel(adj_indices, adj_values, uEmbeds, iEmbeds, uHyper, iHyper):
    n_user = uEmbeds.shape[0]
    e = adj_values.shape[0]

    dst = adj_indices[0]
    order = jnp.argsort(dst)
    dst_s = dst[order]
    src_s = adj_indices[1][order]
    val_s = adj_values[order]

    bounds = jnp.arange(N_WORKERS + 1, dtype=jnp.int32) * RPT
    offs = jnp.searchsorted(dst_s, bounds).astype(jnp.int32)
    counts = offs[1:] - offs[:-1]
    pcount = -(-counts // QUAD) * QUAD
    pstart = jnp.concatenate(
        [jnp.zeros((1,), jnp.int32), jnp.cumsum(pcount)[:-1].astype(jnp.int32)])
    e_cap = e + N_WORKERS * QUAD

    q = jnp.arange(e_cap, dtype=jnp.int32)
    tq = jnp.searchsorted(pstart, q, side="right").astype(jnp.int32) - 1
    rel = q - pstart[tq]
    eidx = jnp.clip(rel + offs[:-1][tq], 0, e - 1)
    valid = rel < counts[tq]
    src_p = jnp.where(valid, src_s[eidx], 0)
    dstl_p = jnp.where(valid, dst_s[eidx] - tq * RPT, 0)
    val_p = jnp.where(valid, val_s[eidx], 0.0)
    valb = jnp.broadcast_to(val_p[:, None], (e_cap, LANES))

    nch = (pcount // C).astype(jnp.int32)
    meta = jnp.concatenate([
        jnp.broadcast_to(nch[:, None], (N_WORKERS, LANES)).reshape(-1),
        jnp.broadcast_to(pstart[:, None], (N_WORKERS, LANES)).reshape(-1),
    ]).astype(jnp.int32)

    x0 = jnp.concatenate([uEmbeds, iEmbeds], axis=0)
    lats = [x0]
    for _ in range(N_LAYERS):
        lats.append(_sc_spmm(lats[-1], src_p, dstl_p, valb, meta, e_cap=e_cap))

    embeds, hyper = _tc_finish(lats[0], lats[1], lats[2], lats[3],
                               uHyper, iHyper)
    return (embeds, hyper[:n_user], hyper[n_user:])


# speedup vs baseline: 2.1257x; 2.1257x over previous
import functools

import jax
import jax.numpy as jnp
from jax import lax
from jax.experimental import pallas as pl
from jax.experimental.pallas import tpu as pltpu
from jax.experimental.pallas import tpu_sc as plsc

D = 256
LANES = 16
C = 32
N_WORKERS = 32
RPT = 320
QUAD = 4 * C
N_LAYERS = 3


def _sc_spmm(table, src, dstl, valb, meta, *, e_cap):
    n_total = table.shape[0]
    mesh = plsc.VectorSubcoreMesh(core_axis_name="c", subcore_axis_name="s")

    @functools.partial(
        pl.kernel,
        out_type=jax.ShapeDtypeStruct((n_total, D), jnp.float32),
        mesh=mesh,
        scratch_types=[
            pltpu.VMEM((RPT, D), jnp.float32),
            pltpu.VMEM((C, D), jnp.float32),
            pltpu.VMEM((C, D), jnp.float32),
            [pltpu.VMEM((C, LANES), jnp.float32) for _ in range(4)],
            [pltpu.VMEM((C,), jnp.int32) for _ in range(4)],
            [pltpu.VMEM((C,), jnp.int32) for _ in range(4)],
            pltpu.VMEM((2 * LANES,), jnp.int32),
            pltpu.SemaphoreType.DMA,
            pltpu.SemaphoreType.DMA,
            [pltpu.SemaphoreType.DMA for _ in range(4)],
            [pltpu.SemaphoreType.DMA for _ in range(4)],
            [pltpu.SemaphoreType.DMA for _ in range(4)],
            pltpu.SemaphoreType.DMA,
        ],
    )
    def spmm_kernel(table_h, src_h, dstl_h, valb_h, meta_h, out_h,
                    acc, g0, g1, vr, sr, dr, metav,
                    gsem0, gsem1, vsem, ssem, dsem, msem):
        core = lax.axis_index("c")
        sub = lax.axis_index("s")
        wid = sub * 2 + core
        gbufs = (g0, g1)
        gsems = (gsem0, gsem1)

        pltpu.async_copy(meta_h.at[pl.ds(wid * LANES, LANES)],
                         metav.at[pl.ds(0, LANES)], msem)
        pltpu.async_copy(meta_h.at[pl.ds((N_WORKERS + wid) * LANES, LANES)],
                         metav.at[pl.ds(LANES, LANES)], msem)

        zero = jnp.zeros((LANES,), jnp.float32)

        def zrow(j, carry):
            for k in range(D // LANES):
                acc[j, pl.ds(k * LANES, LANES)] = zero
            return carry

        lax.fori_loop(0, RPT, zrow, 0)

        pltpu.make_async_copy(meta_h.at[pl.ds(wid * LANES, LANES)],
                              metav.at[pl.ds(0, LANES)], msem).wait()
        pltpu.make_async_copy(meta_h.at[pl.ds((N_WORKERS + wid) * LANES, LANES)],
                              metav.at[pl.ds(LANES, LANES)], msem).wait()
        nch = metav[pl.ds(0, LANES)][0]
        pst = pl.multiple_of(metav[pl.ds(LANES, LANES)][0], QUAD)

        def idx_start(ci, q):
            off = pst + ci * C
            pltpu.async_copy(src_h.at[pl.ds(off, C)], sr[q], ssem[q])
            pltpu.async_copy(dstl_h.at[pl.ds(off, C)], dr[q], dsem[q])
            pltpu.async_copy(valb_h.at[pl.ds(off, C)], vr[q], vsem[q])

        def idx_wait_src(ci, q):
            off = pst + ci * C
            pltpu.make_async_copy(src_h.at[pl.ds(off, C)], sr[q], ssem[q]).wait()

        def idx_wait_rest(ci, q):
            off = pst + ci * C
            pltpu.make_async_copy(dstl_h.at[pl.ds(off, C)], dr[q], dsem[q]).wait()
            pltpu.make_async_copy(valb_h.at[pl.ds(off, C)], vr[q], vsem[q]).wait()

        def rows_start(q, b):
            pltpu.async_copy(table_h.at[sr[q]], gbufs[b], gsems[b])

        def rows_wait(q, b):
            pltpu.make_async_copy(table_h.at[sr[q]], gbufs[b], gsems[b]).wait()

        @pl.when(nch > 0)
        def _():
            for q in range(4):
                idx_start(q, q)
            for q in range(2):
                idx_wait_src(q, q)
                rows_start(q, q)

        def run_flush(row, regs):
            for k in range(D // LANES):
                sl = pl.ds(k * LANES, LANES)
                acc[row, sl] = acc[row, sl] + regs[k]

        def chunk_quad(g, carry):
            ci0 = g * 4
            for u in range(4):
                ci = ci0 + u
                b = u % 2
                gb = gbufs[b]
                rows_wait(u, b)
                idx_wait_rest(ci, u)

                def group(jg, carry2):
                    cur_r, regs = carry2
                    regs = list(regs)
                    j0 = jg * LANES
                    dvec = dr[u][pl.ds(j0, LANES)]
                    for jj in range(LANES):
                        j = j0 + jj
                        r = dvec[jj]
                        vv = vr[u][j]
                        changed = r != cur_r
                        prev_regs = tuple(regs)
                        prev_r = cur_r

                        @pl.when(changed)
                        def _():
                            run_flush(prev_r, prev_regs)

                        regs = [jnp.where(changed, 0.0, regs[k])
                                + gb[j, pl.ds(k * LANES, LANES)] * vv
                                for k in range(D // LANES)]
                        cur_r = r
                    return (cur_r, tuple(regs))

                carry = lax.fori_loop(0, C // LANES, group, carry)

                @pl.when(ci + 4 < nch)
                def _():
                    idx_start(ci + 4, u)

                @pl.when(ci + 2 < nch)
                def _():
                    idx_wait_src(ci + 2, (u + 2) % 4)
                    rows_start((u + 2) % 4, b)
            return carry

        zero16 = tuple(zero for _ in range(D // LANES))
        final_r, final_regs = lax.fori_loop(0, nch // 4, chunk_quad,
                                            (jnp.int32(0), zero16))
        run_flush(final_r, final_regs)

        rbase = wid * RPT

        @pl.when(wid < N_WORKERS - 1)
        def _():
            pltpu.sync_copy(acc.at[pl.ds(0, RPT)], out_h.at[pl.ds(rbase, RPT)])

        @pl.when(wid == N_WORKERS - 1)
        def _():
            last = n_total - (N_WORKERS - 1) * RPT
            pltpu.sync_copy(acc.at[pl.ds(0, last)], out_h.at[pl.ds(rbase, last)])

    return spmm_kernel(table, src, dstl, valb, meta)


def _tc_finish(x0, l1, l2, l3, uHyper, iHyper):
    n_total = x0.shape[0]
    bl = 1000
    nb = n_total // bl
    half_blocks = nb // 2

    def body(x0r, l1r, l2r, l3r, uhr, ihr, emb_r, hyp_r, gscr):
        i = pl.program_id(0)

        @pl.when(i == 0)
        def _():
            gscr[0] = lax.dot_general(uhr[...], uhr[...],
                                      (((0,), (0,)), ((), ())),
                                      preferred_element_type=jnp.float32)
            gscr[1] = lax.dot_general(ihr[...], ihr[...],
                                      (((0,), (0,)), ((), ())),
                                      preferred_element_type=jnp.float32)

        e = x0r[...] + l1r[...] + l2r[...] + l3r[...]
        emb_r[...] = e
        g = jnp.where(i < half_blocks, gscr[0], gscr[1])
        hyp_r[...] = jnp.dot(e, g, preferred_element_type=jnp.float32)

    blk = pl.BlockSpec((bl, D), lambda i: (i, 0))
    full = pl.BlockSpec(uHyper.shape, lambda i: (0, 0))
    return pl.pallas_call(
        body,
        grid=(nb,),
        in_specs=[blk, blk, blk, blk, full, full],
        out_specs=[blk, blk],
        out_shape=[jax.ShapeDtypeStruct((n_total, D), jnp.float32),
                   jax.ShapeDtypeStruct((n_total, D), jnp.float32)],
        scratch_shapes=[pltpu.VMEM((2, D, D), jnp.float32)],
    )(x0, l1, l2, l3, uHyper, iHyper)


def kernel(adj_indices, adj_values, uEmbeds, iEmbeds, uHyper, iHyper):
    n_user = uEmbeds.shape[0]
    e = adj_values.shape[0]

    dst = adj_indices[0]
    order = jnp.argsort(dst)
    dst_s = dst[order]
    src_s = adj_indices[1][order]
    val_s = adj_values[order]

    bounds = jnp.arange(N_WORKERS + 1, dtype=jnp.int32) * RPT
    offs = jnp.searchsorted(dst_s, bounds).astype(jnp.int32)
    counts = offs[1:] - offs[:-1]
    pcount = -(-counts // QUAD) * QUAD
    pstart = jnp.concatenate(
        [jnp.zeros((1,), jnp.int32), jnp.cumsum(pcount)[:-1].astype(jnp.int32)])
    e_cap = e + N_WORKERS * QUAD

    q = jnp.arange(e_cap, dtype=jnp.int32)
    tq = jnp.searchsorted(pstart, q, side="right").astype(jnp.int32) - 1
    rel = q - pstart[tq]
    eidx = jnp.clip(rel + offs[:-1][tq], 0, e - 1)
    valid = rel < counts[tq]
    src_p = jnp.where(valid, src_s[eidx], 0)
    dstl_p = jnp.where(valid, dst_s[eidx] - tq * RPT, 0)
    val_p = jnp.where(valid, val_s[eidx], 0.0)
    valb = jnp.broadcast_to(val_p[:, None], (e_cap, LANES))

    nch = (pcount // C).astype(jnp.int32)
    meta = jnp.concatenate([
        jnp.broadcast_to(nch[:, None], (N_WORKERS, LANES)).reshape(-1),
        jnp.broadcast_to(pstart[:, None], (N_WORKERS, LANES)).reshape(-1),
    ]).astype(jnp.int32)

    x0 = jnp.concatenate([uEmbeds, iEmbeds], axis=0)
    lats = [x0]
    for _ in range(N_LAYERS):
        lats.append(_sc_spmm(lats[-1], src_p, dstl_p, valb, meta, e_cap=e_cap))

    embeds, hyper = _tc_finish(lats[0], lats[1], lats[2], lats[3],
                               uHyper, iHyper)
    return (embeds, hyper[:n_user], hyper[n_user:])


# speedup vs baseline: 2.1312x; 1.0026x over previous
import functools

import jax
import jax.numpy as jnp
from jax import lax
from jax.experimental import pallas as pl
from jax.experimental.pallas import tpu as pltpu
from jax.experimental.pallas import tpu_sc as plsc

D = 256
LANES = 16
C = 32
N_WORKERS = 32
RPT = 320
QUAD = 4 * C
N_LAYERS = 3


def _sc_spmm(table, src, dstl, valb, meta, *, e_cap):
    n_total = table.shape[0]
    mesh = plsc.VectorSubcoreMesh(core_axis_name="c", subcore_axis_name="s")

    @functools.partial(
        pl.kernel,
        out_type=jax.ShapeDtypeStruct((n_total, D), jnp.float32),
        mesh=mesh,
        scratch_types=[
            pltpu.VMEM((RPT, D), jnp.float32),
            pltpu.VMEM((C, D), jnp.float32),
            pltpu.VMEM((C, D), jnp.float32),
            [pltpu.VMEM((C, LANES), jnp.float32) for _ in range(4)],
            [pltpu.VMEM((C,), jnp.int32) for _ in range(4)],
            [pltpu.VMEM((C,), jnp.int32) for _ in range(4)],
            pltpu.VMEM((2 * LANES,), jnp.int32),
            pltpu.SemaphoreType.DMA,
            pltpu.SemaphoreType.DMA,
            [pltpu.SemaphoreType.DMA for _ in range(4)],
            [pltpu.SemaphoreType.DMA for _ in range(4)],
            [pltpu.SemaphoreType.DMA for _ in range(4)],
            pltpu.SemaphoreType.DMA,
        ],
    )
    def spmm_kernel(table_h, src_h, dstl_h, valb_h, meta_h, out_h,
                    acc, g0, g1, vr, sr, dr, metav,
                    gsem0, gsem1, vsem, ssem, dsem, msem):
        core = lax.axis_index("c")
        sub = lax.axis_index("s")
        wid = sub * 2 + core
        gbufs = (g0, g1)
        gsems = (gsem0, gsem1)

        pltpu.async_copy(meta_h.at[pl.ds(wid * LANES, LANES)],
                         metav.at[pl.ds(0, LANES)], msem)
        pltpu.async_copy(meta_h.at[pl.ds((N_WORKERS + wid) * LANES, LANES)],
                         metav.at[pl.ds(LANES, LANES)], msem)

        zero = jnp.zeros((LANES,), jnp.float32)

        def zrow(j, carry):
            for k in range(D // LANES):
                acc[j, pl.ds(k * LANES, LANES)] = zero
            return carry

        lax.fori_loop(0, RPT, zrow, 0)

        pltpu.make_async_copy(meta_h.at[pl.ds(wid * LANES, LANES)],
                              metav.at[pl.ds(0, LANES)], msem).wait()
        pltpu.make_async_copy(meta_h.at[pl.ds((N_WORKERS + wid) * LANES, LANES)],
                              metav.at[pl.ds(LANES, LANES)], msem).wait()
        nch = metav[pl.ds(0, LANES)][0]
        pst = pl.multiple_of(metav[pl.ds(LANES, LANES)][0], QUAD)

        def idx_start(ci, q):
            off = pst + ci * C
            pltpu.async_copy(src_h.at[pl.ds(off, C)], sr[q], ssem[q])
            pltpu.async_copy(dstl_h.at[pl.ds(off, C)], dr[q], dsem[q])
            pltpu.async_copy(valb_h.at[pl.ds(off, C)], vr[q], vsem[q])

        def idx_wait_src(ci, q):
            off = pst + ci * C
            pltpu.make_async_copy(src_h.at[pl.ds(off, C)], sr[q], ssem[q]).wait()

        def idx_wait_rest(ci, q):
            off = pst + ci * C
            pltpu.make_async_copy(dstl_h.at[pl.ds(off, C)], dr[q], dsem[q]).wait()
            pltpu.make_async_copy(valb_h.at[pl.ds(off, C)], vr[q], vsem[q]).wait()

        def rows_start(q, b):
            pltpu.async_copy(table_h.at[sr[q]], gbufs[b], gsems[b])

        def rows_wait(q, b):
            pltpu.make_async_copy(table_h.at[sr[q]], gbufs[b], gsems[b]).wait()

        @pl.when(nch > 0)
        def _():
            for q in range(4):
                idx_start(q, q)
            for q in range(2):
                idx_wait_src(q, q)
                rows_start(q, q)

        def run_flush(row, regs):
            for k in range(D // LANES):
                sl = pl.ds(k * LANES, LANES)
                acc[row, sl] = acc[row, sl] + regs[k]

        def chunk_quad(g, carry):
            ci0 = g * 4
            for u in range(4):
                ci = ci0 + u
                b = u % 2
                gb = gbufs[b]
                rows_wait(u, b)
                idx_wait_rest(ci, u)

                def group(jg, carry2):
                    cur_r, regs = carry2
                    regs = list(regs)
                    j0 = jg * LANES
                    dvec = dr[u][pl.ds(j0, LANES)]
                    for jj in range(LANES):
                        j = j0 + jj
                        r = dvec[jj]
                        vv = vr[u][j]
                        changed = r != cur_r
                        prev_regs = tuple(regs)
                        prev_r = cur_r

                        @pl.when(changed)
                        def _():
                            run_flush(prev_r, prev_regs)

                        regs = [jnp.where(changed, 0.0, regs[k])
                                + gb[j, pl.ds(k * LANES, LANES)] * vv
                                for k in range(D // LANES)]
                        cur_r = r
                    return (cur_r, tuple(regs))

                carry = lax.fori_loop(0, C // LANES, group, carry)

                @pl.when(ci + 4 < nch)
                def _():
                    idx_start(ci + 4, u)

                @pl.when(ci + 2 < nch)
                def _():
                    idx_wait_src(ci + 2, (u + 2) % 4)
                    rows_start((u + 2) % 4, b)
            return carry

        zero16 = tuple(zero for _ in range(D // LANES))
        final_r, final_regs = lax.fori_loop(0, nch // 4, chunk_quad,
                                            (jnp.int32(0), zero16))
        run_flush(final_r, final_regs)

        rbase = wid * RPT

        @pl.when(wid < N_WORKERS - 1)
        def _():
            pltpu.sync_copy(acc.at[pl.ds(0, RPT)], out_h.at[pl.ds(rbase, RPT)])

        @pl.when(wid == N_WORKERS - 1)
        def _():
            last = n_total - (N_WORKERS - 1) * RPT
            pltpu.sync_copy(acc.at[pl.ds(0, last)], out_h.at[pl.ds(rbase, last)])

    return spmm_kernel(table, src, dstl, valb, meta)


def _tc_finish(x0, l1, l2, l3, uHyper, iHyper):
    n_total = x0.shape[0]
    bl = 1000
    nb = n_total // bl
    half_blocks = nb // 2

    def body(x0r, l1r, l2r, l3r, uhr, ihr, emb_r, hyp_r, gscr):
        i = pl.program_id(0)

        @pl.when(i == 0)
        def _():
            gscr[0] = lax.dot_general(uhr[...], uhr[...],
                                      (((0,), (0,)), ((), ())),
                                      preferred_element_type=jnp.float32)
            gscr[1] = lax.dot_general(ihr[...], ihr[...],
                                      (((0,), (0,)), ((), ())),
                                      preferred_element_type=jnp.float32)

        e = x0r[...] + l1r[...] + l2r[...] + l3r[...]
        emb_r[...] = e
        g = jnp.where(i < half_blocks, gscr[0], gscr[1])
        hyp_r[...] = jnp.dot(e, g, preferred_element_type=jnp.float32)

    blk = pl.BlockSpec((bl, D), lambda i: (i, 0))
    full = pl.BlockSpec(uHyper.shape, lambda i: (0, 0))
    return pl.pallas_call(
        body,
        grid=(nb,),
        in_specs=[blk, blk, blk, blk, full, full],
        out_specs=[blk, blk],
        out_shape=[jax.ShapeDtypeStruct((n_total, D), jnp.float32),
                   jax.ShapeDtypeStruct((n_total, D), jnp.float32)],
        scratch_shapes=[pltpu.VMEM((2, D, D), jnp.float32)],
    )(x0, l1, l2, l3, uHyper, iHyper)


def kernel(adj_indices, adj_values, uEmbeds, iEmbeds, uHyper, iHyper):
    n_user = uEmbeds.shape[0]
    e = adj_values.shape[0]

    dst_s, src_s, val_s = lax.sort(
        (adj_indices[0], adj_indices[1], adj_values), num_keys=1)

    bounds = jnp.arange(N_WORKERS + 1, dtype=jnp.int32) * RPT
    offs = jnp.searchsorted(dst_s, bounds).astype(jnp.int32)
    counts = offs[1:] - offs[:-1]
    pcount = -(-counts // QUAD) * QUAD
    pstart = jnp.concatenate(
        [jnp.zeros((1,), jnp.int32), jnp.cumsum(pcount)[:-1].astype(jnp.int32)])
    e_cap = e + N_WORKERS * QUAD

    q = jnp.arange(e_cap, dtype=jnp.int32)
    tq = jnp.searchsorted(pstart, q, side="right").astype(jnp.int32) - 1
    rel = q - pstart[tq]
    eidx = jnp.clip(rel + offs[:-1][tq], 0, e - 1)
    valid = rel < counts[tq]
    src_p = jnp.where(valid, src_s[eidx], 0)
    dstl_p = jnp.where(valid, dst_s[eidx] - tq * RPT, 0)
    val_p = jnp.where(valid, val_s[eidx], 0.0)
    valb = jnp.broadcast_to(val_p[:, None], (e_cap, LANES))

    nch = (pcount // C).astype(jnp.int32)
    meta = jnp.concatenate([
        jnp.broadcast_to(nch[:, None], (N_WORKERS, LANES)).reshape(-1),
        jnp.broadcast_to(pstart[:, None], (N_WORKERS, LANES)).reshape(-1),
    ]).astype(jnp.int32)

    x0 = jnp.concatenate([uEmbeds, iEmbeds], axis=0)
    lats = [x0]
    for _ in range(N_LAYERS):
        lats.append(_sc_spmm(lats[-1], src_p, dstl_p, valb, meta, e_cap=e_cap))

    embeds, hyper = _tc_finish(lats[0], lats[1], lats[2], lats[3],
                               uHyper, iHyper)
    return (embeds, hyper[:n_user], hyper[n_user:])


# speedup vs baseline: 3.2831x; 1.5405x over previous
import functools

import jax
import jax.numpy as jnp
from jax import lax
from jax.experimental import pallas as pl
from jax.experimental.pallas import tpu as pltpu
from jax.experimental.pallas import tpu_sc as plsc

D = 256
LANES = 16
C = 32
N_WORKERS = 32
RPT = 320
QUAD = 4 * C
N_LAYERS = 3


def _sc_spmm(table, src, dstl, valb, meta, *, e_cap):
    n_total = table.shape[0]
    mesh = plsc.VectorSubcoreMesh(core_axis_name="c", subcore_axis_name="s")

    @functools.partial(
        pl.kernel,
        out_type=jax.ShapeDtypeStruct((n_total, D), jnp.float32),
        mesh=mesh,
        scratch_types=[
            pltpu.VMEM((RPT, D), jnp.float32),
            pltpu.VMEM((C, D), jnp.float32),
            pltpu.VMEM((C, D), jnp.float32),
            [pltpu.VMEM((C, LANES), jnp.float32) for _ in range(4)],
            [pltpu.VMEM((C,), jnp.int32) for _ in range(4)],
            [pltpu.VMEM((C,), jnp.int32) for _ in range(4)],
            pltpu.VMEM((2 * LANES,), jnp.int32),
            pltpu.SemaphoreType.DMA,
            pltpu.SemaphoreType.DMA,
            [pltpu.SemaphoreType.DMA for _ in range(4)],
            [pltpu.SemaphoreType.DMA for _ in range(4)],
            [pltpu.SemaphoreType.DMA for _ in range(4)],
            pltpu.SemaphoreType.DMA,
        ],
    )
    def spmm_kernel(table_h, src_h, dstl_h, valb_h, meta_h, out_h,
                    acc, g0, g1, vr, sr, dr, metav,
                    gsem0, gsem1, vsem, ssem, dsem, msem):
        core = lax.axis_index("c")
        sub = lax.axis_index("s")
        wid = sub * 2 + core
        gbufs = (g0, g1)
        gsems = (gsem0, gsem1)

        pltpu.async_copy(meta_h.at[pl.ds(wid * LANES, LANES)],
                         metav.at[pl.ds(0, LANES)], msem)
        pltpu.async_copy(meta_h.at[pl.ds((N_WORKERS + wid) * LANES, LANES)],
                         metav.at[pl.ds(LANES, LANES)], msem)

        zero = jnp.zeros((LANES,), jnp.float32)

        def zrow(j, carry):
            for k in range(D // LANES):
                acc[j, pl.ds(k * LANES, LANES)] = zero
            return carry

        lax.fori_loop(0, RPT, zrow, 0)

        pltpu.make_async_copy(meta_h.at[pl.ds(wid * LANES, LANES)],
                              metav.at[pl.ds(0, LANES)], msem).wait()
        pltpu.make_async_copy(meta_h.at[pl.ds((N_WORKERS + wid) * LANES, LANES)],
                              metav.at[pl.ds(LANES, LANES)], msem).wait()
        nch = metav[pl.ds(0, LANES)][0]
        pst = pl.multiple_of(metav[pl.ds(LANES, LANES)][0], 8)
        rbase = wid * RPT

        def idx_start(ci, q):
            off = pst + ci * C
            pltpu.async_copy(src_h.at[pl.ds(off, C)], sr[q], ssem[q])
            pltpu.async_copy(dstl_h.at[pl.ds(off, C)], dr[q], dsem[q])
            pltpu.async_copy(valb_h.at[pl.ds(off, C)], vr[q], vsem[q])

        def idx_wait_src(ci, q):
            off = pst + ci * C
            pltpu.make_async_copy(src_h.at[pl.ds(off, C)], sr[q], ssem[q]).wait()

        def idx_wait_rest(ci, q):
            off = pst + ci * C
            pltpu.make_async_copy(dstl_h.at[pl.ds(off, C)], dr[q], dsem[q]).wait()
            pltpu.make_async_copy(valb_h.at[pl.ds(off, C)], vr[q], vsem[q]).wait()

        def rows_start(q, b):
            pltpu.async_copy(table_h.at[sr[q]], gbufs[b], gsems[b])

        def rows_wait(q, b):
            pltpu.make_async_copy(table_h.at[sr[q]], gbufs[b], gsems[b]).wait()

        @pl.when(nch > 0)
        def _():
            for q in range(4):
                idx_start(q, q)
            for q in range(2):
                idx_wait_src(q, q)
                rows_start(q, q)

        def run_flush(row, regs):
            for k in range(D // LANES):
                sl = pl.ds(k * LANES, LANES)
                acc[row, sl] = acc[row, sl] + regs[k]

        def chunk_quad(g, carry):
            ci0 = g * 4
            for u in range(4):
                ci = ci0 + u
                b = u % 2
                gb = gbufs[b]
                rows_wait(u, b)
                idx_wait_rest(ci, u)

                def group(jg, carry2):
                    cur_r, regs = carry2
                    regs = list(regs)
                    j0 = jg * LANES
                    dvec = dr[u][pl.ds(j0, LANES)]
                    rvec = dvec - rbase
                    ok = (rvec >= 0) & (rvec < RPT)
                    rcl = jnp.where(ok, rvec, 0)
                    vmask = jnp.where(ok, 1.0, 0.0)
                    for jj in range(LANES):
                        j = j0 + jj
                        r = rcl[jj]
                        vv = vr[u][j] * vmask[jj]
                        changed = r != cur_r
                        prev_regs = tuple(regs)
                        prev_r = cur_r

                        @pl.when(changed)
                        def _():
                            run_flush(prev_r, prev_regs)

                        regs = [jnp.where(changed, 0.0, regs[k])
                                + gb[j, pl.ds(k * LANES, LANES)] * vv
                                for k in range(D // LANES)]
                        cur_r = r
                    return (cur_r, tuple(regs))

                carry = lax.fori_loop(0, C // LANES, group, carry)

                @pl.when(ci + 4 < nch)
                def _():
                    idx_start(ci + 4, u)

                @pl.when(ci + 2 < nch)
                def _():
                    idx_wait_src(ci + 2, (u + 2) % 4)
                    rows_start((u + 2) % 4, b)
            return carry

        zero16 = tuple(zero for _ in range(D // LANES))
        final_r, final_regs = lax.fori_loop(0, nch // 4, chunk_quad,
                                            (jnp.int32(0), zero16))
        run_flush(final_r, final_regs)

        rbase = wid * RPT

        @pl.when(wid < N_WORKERS - 1)
        def _():
            pltpu.sync_copy(acc.at[pl.ds(0, RPT)], out_h.at[pl.ds(rbase, RPT)])

        @pl.when(wid == N_WORKERS - 1)
        def _():
            last = n_total - (N_WORKERS - 1) * RPT
            pltpu.sync_copy(acc.at[pl.ds(0, last)], out_h.at[pl.ds(rbase, last)])

    return spmm_kernel(table, src, dstl, valb, meta)


def _tc_finish(x0, l1, l2, l3, uHyper, iHyper):
    n_total = x0.shape[0]
    bl = 1000
    nb = n_total // bl
    half_blocks = nb // 2

    def body(x0r, l1r, l2r, l3r, uhr, ihr, emb_r, hyp_r, gscr):
        i = pl.program_id(0)

        @pl.when(i == 0)
        def _():
            gscr[0] = lax.dot_general(uhr[...], uhr[...],
                                      (((0,), (0,)), ((), ())),
                                      preferred_element_type=jnp.float32)
            gscr[1] = lax.dot_general(ihr[...], ihr[...],
                                      (((0,), (0,)), ((), ())),
                                      preferred_element_type=jnp.float32)

        e = x0r[...] + l1r[...] + l2r[...] + l3r[...]
        emb_r[...] = e
        g = jnp.where(i < half_blocks, gscr[0], gscr[1])
        hyp_r[...] = jnp.dot(e, g, preferred_element_type=jnp.float32)

    blk = pl.BlockSpec((bl, D), lambda i: (i, 0))
    full = pl.BlockSpec(uHyper.shape, lambda i: (0, 0))
    return pl.pallas_call(
        body,
        grid=(nb,),
        in_specs=[blk, blk, blk, blk, full, full],
        out_specs=[blk, blk],
        out_shape=[jax.ShapeDtypeStruct((n_total, D), jnp.float32),
                   jax.ShapeDtypeStruct((n_total, D), jnp.float32)],
        scratch_shapes=[pltpu.VMEM((2, D, D), jnp.float32)],
    )(x0, l1, l2, l3, uHyper, iHyper)


def kernel(adj_indices, adj_values, uEmbeds, iEmbeds, uHyper, iHyper):
    n_user = uEmbeds.shape[0]
    e = adj_values.shape[0]

    dst_s, src_s, val_s = lax.sort(
        (adj_indices[0], adj_indices[1], adj_values), num_keys=1)

    pad = 2 * QUAD
    e_cap = e + pad
    zpad_i = jnp.zeros((pad,), jnp.int32)
    src_p = jnp.concatenate([src_s, zpad_i])
    dstl_p = jnp.concatenate([dst_s, zpad_i])
    val_p = jnp.concatenate([val_s, jnp.zeros((pad,), jnp.float32)])
    valb = jnp.broadcast_to(val_p[:, None], (e_cap, LANES))

    bounds = jnp.arange(N_WORKERS + 1, dtype=jnp.int32) * RPT
    offs = jnp.sum(dst_s[:, None] < bounds[None, :], axis=0,
                   dtype=jnp.int32)
    start8 = (offs[:-1] // 8) * 8
    nch = -(-(offs[1:] - start8) // C)
    nch4 = (-(-nch // 4) * 4).astype(jnp.int32)
    meta = jnp.concatenate([
        jnp.broadcast_to(nch4[:, None], (N_WORKERS, LANES)).reshape(-1),
        jnp.broadcast_to(start8[:, None], (N_WORKERS, LANES)).reshape(-1),
    ]).astype(jnp.int32)

    x0 = jnp.concatenate([uEmbeds, iEmbeds], axis=0)
    lats = [x0]
    for _ in range(N_LAYERS):
        lats.append(_sc_spmm(lats[-1], src_p, dstl_p, valb, meta, e_cap=e_cap))

    embeds, hyper = _tc_finish(lats[0], lats[1], lats[2], lats[3],
                               uHyper, iHyper)
    return (embeds, hyper[:n_user], hyper[n_user:])


# speedup vs baseline: 4.0682x; 1.2391x over previous
import functools

import jax
import jax.numpy as jnp
from jax import lax
from jax.experimental import pallas as pl
from jax.experimental.pallas import tpu as pltpu
from jax.experimental.pallas import tpu_sc as plsc

D = 256
LANES = 16
C = 48
N_WORKERS = 32
RPT = 320
QUAD = 4 * C
N_LAYERS = 3


def _sc_spmm(table, src, dstl, valb, meta, *, e_cap):
    n_total = table.shape[0]
    mesh = plsc.VectorSubcoreMesh(core_axis_name="c", subcore_axis_name="s")

    @functools.partial(
        pl.kernel,
        out_type=jax.ShapeDtypeStruct((n_total, D), jnp.float32),
        mesh=mesh,
        scratch_types=[
            pltpu.VMEM((RPT, D), jnp.float32),
            pltpu.VMEM((C, D), jnp.float32),
            pltpu.VMEM((C, D), jnp.float32),
            [pltpu.VMEM((C,), jnp.float32) for _ in range(4)],
            [pltpu.VMEM((C,), jnp.int32) for _ in range(4)],
            [pltpu.VMEM((C,), jnp.int32) for _ in range(4)],
            pltpu.VMEM((2 * LANES,), jnp.int32),
            pltpu.SemaphoreType.DMA,
            pltpu.SemaphoreType.DMA,
            [pltpu.SemaphoreType.DMA for _ in range(4)],
            [pltpu.SemaphoreType.DMA for _ in range(4)],
            [pltpu.SemaphoreType.DMA for _ in range(4)],
            pltpu.SemaphoreType.DMA,
        ],
    )
    def spmm_kernel(table_h, src_h, dstl_h, valb_h, meta_h, out_h,
                    acc, g0, g1, vr, sr, dr, metav,
                    gsem0, gsem1, vsem, ssem, dsem, msem):
        core = lax.axis_index("c")
        sub = lax.axis_index("s")
        wid = sub * 2 + core
        gbufs = (g0, g1)
        gsems = (gsem0, gsem1)

        pltpu.async_copy(meta_h.at[pl.ds(wid * LANES, LANES)],
                         metav.at[pl.ds(0, LANES)], msem)
        pltpu.async_copy(meta_h.at[pl.ds((N_WORKERS + wid) * LANES, LANES)],
                         metav.at[pl.ds(LANES, LANES)], msem)

        zero = jnp.zeros((LANES,), jnp.float32)

        def zrow(j, carry):
            for k in range(D // LANES):
                acc[j, pl.ds(k * LANES, LANES)] = zero
            return carry

        lax.fori_loop(0, RPT, zrow, 0)

        pltpu.make_async_copy(meta_h.at[pl.ds(wid * LANES, LANES)],
                              metav.at[pl.ds(0, LANES)], msem).wait()
        pltpu.make_async_copy(meta_h.at[pl.ds((N_WORKERS + wid) * LANES, LANES)],
                              metav.at[pl.ds(LANES, LANES)], msem).wait()
        nch = metav[pl.ds(0, LANES)][0]
        pst = pl.multiple_of(metav[pl.ds(LANES, LANES)][0], 8)
        rbase = wid * RPT

        def idx_start(ci, q):
            off = pst + ci * C
            pltpu.async_copy(src_h.at[pl.ds(off, C)], sr[q], ssem[q])
            pltpu.async_copy(dstl_h.at[pl.ds(off, C)], dr[q], dsem[q])
            pltpu.async_copy(valb_h.at[pl.ds(off, C)], vr[q], vsem[q])

        def idx_wait_src(ci, q):
            off = pst + ci * C
            pltpu.make_async_copy(src_h.at[pl.ds(off, C)], sr[q], ssem[q]).wait()

        def idx_wait_rest(ci, q):
            off = pst + ci * C
            pltpu.make_async_copy(dstl_h.at[pl.ds(off, C)], dr[q], dsem[q]).wait()
            pltpu.make_async_copy(valb_h.at[pl.ds(off, C)], vr[q], vsem[q]).wait()

        def rows_start(q, b):
            pltpu.async_copy(table_h.at[sr[q]], gbufs[b], gsems[b])

        def rows_wait(q, b):
            pltpu.make_async_copy(table_h.at[sr[q]], gbufs[b], gsems[b]).wait()

        @pl.when(nch > 0)
        def _():
            for q in range(4):
                idx_start(q, q)
            for q in range(2):
                idx_wait_src(q, q)
                rows_start(q, q)

        def run_flush(row, regs):
            for k in range(D // LANES):
                sl = pl.ds(k * LANES, LANES)
                acc[row, sl] = acc[row, sl] + regs[k]

        def chunk_quad(g, carry):
            ci0 = g * 4
            for u in range(4):
                ci = ci0 + u
                b = u % 2
                gb = gbufs[b]
                rows_wait(u, b)
                idx_wait_rest(ci, u)

                def group(jg, carry2):
                    cur_r, regs = carry2
                    regs = list(regs)
                    j0 = jg * LANES
                    dvec = dr[u][pl.ds(j0, LANES)]
                    rvec = dvec - rbase
                    ok = (rvec >= 0) & (rvec < RPT)
                    rcl = jnp.where(ok, rvec, 0)
                    vgrp = jnp.where(ok, vr[u][pl.ds(j0, LANES)], 0.0)
                    for jj in range(LANES):
                        j = j0 + jj
                        r = rcl[jj]
                        vv = vgrp[jj]
                        changed = r != cur_r
                        prev_regs = tuple(regs)
                        prev_r = cur_r

                        @pl.when(changed)
                        def _():
                            run_flush(prev_r, prev_regs)

                        regs = [jnp.where(changed, 0.0, regs[k])
                                + gb[j, pl.ds(k * LANES, LANES)] * vv
                                for k in range(D // LANES)]
                        cur_r = r
                    return (cur_r, tuple(regs))

                carry = lax.fori_loop(0, C // LANES, group, carry)

                @pl.when(ci + 4 < nch)
                def _():
                    idx_start(ci + 4, u)

                @pl.when(ci + 2 < nch)
                def _():
                    idx_wait_src(ci + 2, (u + 2) % 4)
                    rows_start((u + 2) % 4, b)
            return carry

        zero16 = tuple(zero for _ in range(D // LANES))
        final_r, final_regs = lax.fori_loop(0, nch // 4, chunk_quad,
                                            (jnp.int32(0), zero16))
        run_flush(final_r, final_regs)

        rbase = wid * RPT

        @pl.when(wid < N_WORKERS - 1)
        def _():
            pltpu.sync_copy(acc.at[pl.ds(0, RPT)], out_h.at[pl.ds(rbase, RPT)])

        @pl.when(wid == N_WORKERS - 1)
        def _():
            last = n_total - (N_WORKERS - 1) * RPT
            pltpu.sync_copy(acc.at[pl.ds(0, last)], out_h.at[pl.ds(rbase, last)])

    return spmm_kernel(table, src, dstl, valb, meta)


def _tc_finish(x0, l1, l2, l3, uHyper, iHyper):
    n_total = x0.shape[0]
    bl = 1000
    nb = n_total // bl
    half_blocks = nb // 2

    def body(x0r, l1r, l2r, l3r, uhr, ihr, emb_r, hyp_r, gscr):
        i = pl.program_id(0)

        @pl.when(i == 0)
        def _():
            gscr[0] = lax.dot_general(uhr[...], uhr[...],
                                      (((0,), (0,)), ((), ())),
                                      preferred_element_type=jnp.float32)
            gscr[1] = lax.dot_general(ihr[...], ihr[...],
                                      (((0,), (0,)), ((), ())),
                                      preferred_element_type=jnp.float32)

        e = x0r[...] + l1r[...] + l2r[...] + l3r[...]
        emb_r[...] = e
        g = jnp.where(i < half_blocks, gscr[0], gscr[1])
        hyp_r[...] = jnp.dot(e, g, preferred_element_type=jnp.float32)

    blk = pl.BlockSpec((bl, D), lambda i: (i, 0))
    full = pl.BlockSpec(uHyper.shape, lambda i: (0, 0))
    return pl.pallas_call(
        body,
        grid=(nb,),
        in_specs=[blk, blk, blk, blk, full, full],
        out_specs=[blk, blk],
        out_shape=[jax.ShapeDtypeStruct((n_total, D), jnp.float32),
                   jax.ShapeDtypeStruct((n_total, D), jnp.float32)],
        scratch_shapes=[pltpu.VMEM((2, D, D), jnp.float32)],
    )(x0, l1, l2, l3, uHyper, iHyper)


def kernel(adj_indices, adj_values, uEmbeds, iEmbeds, uHyper, iHyper):
    n_user = uEmbeds.shape[0]
    e = adj_values.shape[0]

    dst_s, src_s, val_s = lax.sort(
        (adj_indices[0], adj_indices[1], adj_values), num_keys=1)

    pad = 2 * QUAD
    e_cap = e + pad
    zpad_i = jnp.zeros((pad,), jnp.int32)
    src_p = jnp.concatenate([src_s, zpad_i])
    dstl_p = jnp.concatenate([dst_s, zpad_i])
    val_p = jnp.concatenate([val_s, jnp.zeros((pad,), jnp.float32)])

    bounds = jnp.arange(N_WORKERS + 1, dtype=jnp.int32) * RPT
    offs = jnp.sum(dst_s[:, None] < bounds[None, :], axis=0,
                   dtype=jnp.int32)
    start8 = (offs[:-1] // 8) * 8
    nch = -(-(offs[1:] - start8) // C)
    nch4 = (-(-nch // 4) * 4).astype(jnp.int32)
    meta = jnp.concatenate([
        jnp.broadcast_to(nch4[:, None], (N_WORKERS, LANES)).reshape(-1),
        jnp.broadcast_to(start8[:, None], (N_WORKERS, LANES)).reshape(-1),
    ]).astype(jnp.int32)

    x0 = jnp.concatenate([uEmbeds, iEmbeds], axis=0)
    lats = [x0]
    for _ in range(N_LAYERS):
        lats.append(_sc_spmm(lats[-1], src_p, dstl_p, val_p, meta, e_cap=e_cap))

    embeds, hyper = _tc_finish(lats[0], lats[1], lats[2], lats[3],
                               uHyper, iHyper)
    return (embeds, hyper[:n_user], hyper[n_user:])


# speedup vs baseline: 4.3759x; 1.0756x over previous
import functools

import jax
import jax.numpy as jnp
from jax import lax
from jax.experimental import pallas as pl
from jax.experimental.pallas import tpu as pltpu
from jax.experimental.pallas import tpu_sc as plsc

D = 256
LANES = 16
C = 64
N_WORKERS = 32
RPT = 320
QUAD = 4 * C
N_LAYERS = 3


def _sc_spmm(table, src, dstl, valb, meta, *, e_cap):
    n_total = table.shape[0]
    mesh = plsc.VectorSubcoreMesh(core_axis_name="c", subcore_axis_name="s")

    @functools.partial(
        pl.kernel,
        out_type=jax.ShapeDtypeStruct((n_total, D), jnp.float32),
        mesh=mesh,
        scratch_types=[
            pltpu.VMEM((RPT, D), jnp.float32),
            pltpu.VMEM((C, D), jnp.float32),
            pltpu.VMEM((C, D), jnp.float32),
            [pltpu.VMEM((C,), jnp.float32) for _ in range(4)],
            [pltpu.VMEM((C,), jnp.int32) for _ in range(4)],
            [pltpu.VMEM((C,), jnp.int32) for _ in range(4)],
            pltpu.VMEM((2 * LANES,), jnp.int32),
            pltpu.SemaphoreType.DMA,
            pltpu.SemaphoreType.DMA,
            [pltpu.SemaphoreType.DMA for _ in range(4)],
            [pltpu.SemaphoreType.DMA for _ in range(4)],
            [pltpu.SemaphoreType.DMA for _ in range(4)],
            pltpu.SemaphoreType.DMA,
        ],
    )
    def spmm_kernel(table_h, src_h, dstl_h, valb_h, meta_h, out_h,
                    acc, g0, g1, vr, sr, dr, metav,
                    gsem0, gsem1, vsem, ssem, dsem, msem):
        core = lax.axis_index("c")
        sub = lax.axis_index("s")
        wid = sub * 2 + core
        gbufs = (g0, g1)
        gsems = (gsem0, gsem1)

        pltpu.async_copy(meta_h.at[pl.ds(wid * LANES, LANES)],
                         metav.at[pl.ds(0, LANES)], msem)
        pltpu.async_copy(meta_h.at[pl.ds((N_WORKERS + wid) * LANES, LANES)],
                         metav.at[pl.ds(LANES, LANES)], msem)

        zero = jnp.zeros((LANES,), jnp.float32)

        def zrow(j, carry):
            for k in range(D // LANES):
                acc[j, pl.ds(k * LANES, LANES)] = zero
            return carry

        lax.fori_loop(0, RPT, zrow, 0)

        pltpu.make_async_copy(meta_h.at[pl.ds(wid * LANES, LANES)],
                              metav.at[pl.ds(0, LANES)], msem).wait()
        pltpu.make_async_copy(meta_h.at[pl.ds((N_WORKERS + wid) * LANES, LANES)],
                              metav.at[pl.ds(LANES, LANES)], msem).wait()
        nch = metav[pl.ds(0, LANES)][0]
        pst = pl.multiple_of(metav[pl.ds(LANES, LANES)][0], 8)
        rbase = wid * RPT

        def idx_start(ci, q):
            off = pst + ci * C
            pltpu.async_copy(src_h.at[pl.ds(off, C)], sr[q], ssem[q])
            pltpu.async_copy(dstl_h.at[pl.ds(off, C)], dr[q], dsem[q])
            pltpu.async_copy(valb_h.at[pl.ds(off, C)], vr[q], vsem[q])

        def idx_wait_src(ci, q):
            off = pst + ci * C
            pltpu.make_async_copy(src_h.at[pl.ds(off, C)], sr[q], ssem[q]).wait()

        def idx_wait_rest(ci, q):
            off = pst + ci * C
            pltpu.make_async_copy(dstl_h.at[pl.ds(off, C)], dr[q], dsem[q]).wait()
            pltpu.make_async_copy(valb_h.at[pl.ds(off, C)], vr[q], vsem[q]).wait()

        def rows_start(q, b):
            pltpu.async_copy(table_h.at[sr[q]], gbufs[b], gsems[b])

        def rows_wait(q, b):
            pltpu.make_async_copy(table_h.at[sr[q]], gbufs[b], gsems[b]).wait()

        @pl.when(nch > 0)
        def _():
            for q in range(4):
                idx_start(q, q)
            for q in range(2):
                idx_wait_src(q, q)
                rows_start(q, q)

        def run_flush(row, regs):
            for k in range(D // LANES):
                sl = pl.ds(k * LANES, LANES)
                acc[row, sl] = acc[row, sl] + regs[k]

        def chunk_quad(g, carry):
            ci0 = g * 4
            for u in range(4):
                ci = ci0 + u
                b = u % 2
                gb = gbufs[b]
                rows_wait(u, b)
                idx_wait_rest(ci, u)

                def group(jg, carry2):
                    cur_r, regs = carry2
                    regs = list(regs)
                    j0 = jg * LANES
                    dvec = dr[u][pl.ds(j0, LANES)]
                    rvec = dvec - rbase
                    ok = (rvec >= 0) & (rvec < RPT)
                    rcl = jnp.where(ok, rvec, 0)
                    vgrp = jnp.where(ok, vr[u][pl.ds(j0, LANES)], 0.0)
                    for jj in range(LANES):
                        j = j0 + jj
                        r = rcl[jj]
                        vv = vgrp[jj]
                        changed = r != cur_r
                        prev_regs = tuple(regs)
                        prev_r = cur_r

                        @pl.when(changed)
                        def _():
                            run_flush(prev_r, prev_regs)

                        regs = [jnp.where(changed, 0.0, regs[k])
                                + gb[j, pl.ds(k * LANES, LANES)] * vv
                                for k in range(D // LANES)]
                        cur_r = r
                    return (cur_r, tuple(regs))

                carry = lax.fori_loop(0, C // LANES, group, carry)

                @pl.when(ci + 4 < nch)
                def _():
                    idx_start(ci + 4, u)

                @pl.when(ci + 2 < nch)
                def _():
                    idx_wait_src(ci + 2, (u + 2) % 4)
                    rows_start((u + 2) % 4, b)
            return carry

        zero16 = tuple(zero for _ in range(D // LANES))
        final_r, final_regs = lax.fori_loop(0, nch // 4, chunk_quad,
                                            (jnp.int32(0), zero16))
        run_flush(final_r, final_regs)

        rbase = wid * RPT

        @pl.when(wid < N_WORKERS - 1)
        def _():
            pltpu.sync_copy(acc.at[pl.ds(0, RPT)], out_h.at[pl.ds(rbase, RPT)])

        @pl.when(wid == N_WORKERS - 1)
        def _():
            last = n_total - (N_WORKERS - 1) * RPT
            pltpu.sync_copy(acc.at[pl.ds(0, last)], out_h.at[pl.ds(rbase, last)])

    return spmm_kernel(table, src, dstl, valb, meta)


def _tc_finish(x0, l1, l2, l3, uHyper, iHyper):
    n_total = x0.shape[0]
    bl = 1000
    nb = n_total // bl
    half_blocks = nb // 2

    def body(x0r, l1r, l2r, l3r, uhr, ihr, emb_r, hyp_r, gscr):
        i = pl.program_id(0)

        @pl.when(i == 0)
        def _():
            gscr[0] = lax.dot_general(uhr[...], uhr[...],
                                      (((0,), (0,)), ((), ())),
                                      preferred_element_type=jnp.float32)
            gscr[1] = lax.dot_general(ihr[...], ihr[...],
                                      (((0,), (0,)), ((), ())),
                                      preferred_element_type=jnp.float32)

        e = x0r[...] + l1r[...] + l2r[...] + l3r[...]
        emb_r[...] = e
        g = jnp.where(i < half_blocks, gscr[0], gscr[1])
        hyp_r[...] = jnp.dot(e, g, preferred_element_type=jnp.float32)

    blk = pl.BlockSpec((bl, D), lambda i: (i, 0))
    full = pl.BlockSpec(uHyper.shape, lambda i: (0, 0))
    return pl.pallas_call(
        body,
        grid=(nb,),
        in_specs=[blk, blk, blk, blk, full, full],
        out_specs=[blk, blk],
        out_shape=[jax.ShapeDtypeStruct((n_total, D), jnp.float32),
                   jax.ShapeDtypeStruct((n_total, D), jnp.float32)],
        scratch_shapes=[pltpu.VMEM((2, D, D), jnp.float32)],
    )(x0, l1, l2, l3, uHyper, iHyper)


def kernel(adj_indices, adj_values, uEmbeds, iEmbeds, uHyper, iHyper):
    n_user = uEmbeds.shape[0]
    e = adj_values.shape[0]

    dst_s, src_s, val_s = lax.sort(
        (adj_indices[0], adj_indices[1], adj_values), num_keys=1)

    pad = 2 * QUAD
    e_cap = e + pad
    zpad_i = jnp.zeros((pad,), jnp.int32)
    src_p = jnp.concatenate([src_s, zpad_i])
    dstl_p = jnp.concatenate([dst_s, zpad_i])
    val_p = jnp.concatenate([val_s, jnp.zeros((pad,), jnp.float32)])

    bounds = jnp.arange(N_WORKERS + 1, dtype=jnp.int32) * RPT
    offs = jnp.sum(dst_s[:, None] < bounds[None, :], axis=0,
                   dtype=jnp.int32)
    start8 = (offs[:-1] // 8) * 8
    nch = -(-(offs[1:] - start8) // C)
    nch4 = (-(-nch // 4) * 4).astype(jnp.int32)
    meta = jnp.concatenate([
        jnp.broadcast_to(nch4[:, None], (N_WORKERS, LANES)).reshape(-1),
        jnp.broadcast_to(start8[:, None], (N_WORKERS, LANES)).reshape(-1),
    ]).astype(jnp.int32)

    x0 = jnp.concatenate([uEmbeds, iEmbeds], axis=0)
    lats = [x0]
    for _ in range(N_LAYERS):
        lats.append(_sc_spmm(lats[-1], src_p, dstl_p, val_p, meta, e_cap=e_cap))

    embeds, hyper = _tc_finish(lats[0], lats[1], lats[2], lats[3],
                               uHyper, iHyper)
    return (embeds, hyper[:n_user], hyper[n_user:])


# speedup vs baseline: 4.6393x; 1.0602x over previous
import functools

import jax
import jax.numpy as jnp
from jax import lax
from jax.experimental import pallas as pl
from jax.experimental.pallas import tpu as pltpu
from jax.experimental.pallas import tpu_sc as plsc

D = 256
LANES = 16
C = 64
N_WORKERS = 32
RPT = 320
QUAD = 4 * C
N_LAYERS = 3


def _sc_spmm(table, src, dstl, valb, meta, *, e_cap):
    n_total = table.shape[0]
    mesh = plsc.VectorSubcoreMesh(core_axis_name="c", subcore_axis_name="s")

    @functools.partial(
        pl.kernel,
        out_type=jax.ShapeDtypeStruct((n_total, D), jnp.float32),
        mesh=mesh,
        scratch_types=[
            pltpu.VMEM((RPT, D), jnp.float32),
            pltpu.VMEM((C, D), jnp.float32),
            pltpu.VMEM((C, D), jnp.float32),
            [pltpu.VMEM((C,), jnp.float32) for _ in range(4)],
            [pltpu.VMEM((C,), jnp.int32) for _ in range(4)],
            [pltpu.VMEM((C,), jnp.int32) for _ in range(4)],
            pltpu.VMEM((2 * LANES,), jnp.int32),
            pltpu.SemaphoreType.DMA,
            pltpu.SemaphoreType.DMA,
            [pltpu.SemaphoreType.DMA for _ in range(4)],
            [pltpu.SemaphoreType.DMA for _ in range(4)],
            [pltpu.SemaphoreType.DMA for _ in range(4)],
            pltpu.SemaphoreType.DMA,
        ],
    )
    def spmm_kernel(table_h, src_h, dstl_h, valb_h, meta_h, out_h,
                    acc, g0, g1, vr, sr, dr, metav,
                    gsem0, gsem1, vsem, ssem, dsem, msem):
        core = lax.axis_index("c")
        sub = lax.axis_index("s")
        wid = sub * 2 + core
        gbufs = (g0, g1)
        gsems = (gsem0, gsem1)

        pltpu.async_copy(meta_h.at[pl.ds(wid * LANES, LANES)],
                         metav.at[pl.ds(0, LANES)], msem)
        pltpu.async_copy(meta_h.at[pl.ds((N_WORKERS + wid) * LANES, LANES)],
                         metav.at[pl.ds(LANES, LANES)], msem)

        zero = jnp.zeros((LANES,), jnp.float32)

        def zrow(j, carry):
            for k in range(D // LANES):
                acc[j, pl.ds(k * LANES, LANES)] = zero
            return carry

        lax.fori_loop(0, RPT, zrow, 0)

        pltpu.make_async_copy(meta_h.at[pl.ds(wid * LANES, LANES)],
                              metav.at[pl.ds(0, LANES)], msem).wait()
        pltpu.make_async_copy(meta_h.at[pl.ds((N_WORKERS + wid) * LANES, LANES)],
                              metav.at[pl.ds(LANES, LANES)], msem).wait()
        nch = metav[pl.ds(0, LANES)][0]
        pst = pl.multiple_of(metav[pl.ds(LANES, LANES)][0], 8)
        rbase = wid * RPT

        def idx_start(ci, q):
            off = pst + ci * C
            pltpu.async_copy(src_h.at[pl.ds(off, C)], sr[q], ssem[q])
            pltpu.async_copy(dstl_h.at[pl.ds(off, C)], dr[q], dsem[q])
            pltpu.async_copy(valb_h.at[pl.ds(off, C)], vr[q], vsem[q])

        def idx_wait_src(ci, q):
            off = pst + ci * C
            pltpu.make_async_copy(src_h.at[pl.ds(off, C)], sr[q], ssem[q]).wait()

        def idx_wait_rest(ci, q):
            off = pst + ci * C
            pltpu.make_async_copy(dstl_h.at[pl.ds(off, C)], dr[q], dsem[q]).wait()
            pltpu.make_async_copy(valb_h.at[pl.ds(off, C)], vr[q], vsem[q]).wait()

        def rows_start(q, b):
            pltpu.async_copy(table_h.at[sr[q]], gbufs[b], gsems[b])

        def rows_wait(q, b):
            pltpu.make_async_copy(table_h.at[sr[q]], gbufs[b], gsems[b]).wait()

        @pl.when(nch > 0)
        def _():
            for q in range(4):
                idx_start(q, q)
            for q in range(2):
                idx_wait_src(q, q)
                rows_start(q, q)

        def run_flush(row, regs):
            for k in range(D // LANES):
                sl = pl.ds(k * LANES, LANES)
                acc[row, sl] = acc[row, sl] + regs[k]

        def chunk_quad(g, carry):
            ci0 = g * 4
            for u in range(4):
                ci = ci0 + u
                b = u % 2
                gb = gbufs[b]
                rows_wait(u, b)
                idx_wait_rest(ci, u)

                def group(jg, carry2):
                    cur_r, regs = carry2
                    regs = list(regs)
                    j0 = jg * LANES
                    dvec = dr[u][pl.ds(j0, LANES)]
                    rvec = dvec - rbase
                    ok = (rvec >= 0) & (rvec < RPT)
                    rcl = jnp.where(ok, rvec, 0)
                    vgrp = jnp.where(ok, vr[u][pl.ds(j0, LANES)], 0.0)
                    for jj in range(LANES):
                        j = j0 + jj
                        r = rcl[jj]
                        vv = vgrp[jj]
                        changed = r != cur_r
                        prev_regs = tuple(regs)
                        prev_r = cur_r

                        @pl.when(changed)
                        def _():
                            run_flush(prev_r, prev_regs)

                        regs = [jnp.where(changed, 0.0, regs[k])
                                + gb[j, pl.ds(k * LANES, LANES)] * vv
                                for k in range(D // LANES)]
                        cur_r = r
                    return (cur_r, tuple(regs))

                carry = lax.fori_loop(0, C // LANES, group, carry)

                @pl.when(ci + 4 < nch)
                def _():
                    idx_start(ci + 4, u)

                @pl.when(ci + 2 < nch)
                def _():
                    idx_wait_src(ci + 2, (u + 2) % 4)
                    rows_start((u + 2) % 4, b)
            return carry

        zero16 = tuple(zero for _ in range(D // LANES))
        final_r, final_regs = lax.fori_loop(0, nch // 4, chunk_quad,
                                            (jnp.int32(0), zero16))
        run_flush(final_r, final_regs)

        rbase = wid * RPT

        @pl.when(wid < N_WORKERS - 1)
        def _():
            pltpu.sync_copy(acc.at[pl.ds(0, RPT)], out_h.at[pl.ds(rbase, RPT)])

        @pl.when(wid == N_WORKERS - 1)
        def _():
            last = n_total - (N_WORKERS - 1) * RPT
            pltpu.sync_copy(acc.at[pl.ds(0, last)], out_h.at[pl.ds(rbase, last)])

    return spmm_kernel(table, src, dstl, valb, meta)


def _tc_finish(x0, l1, l2, l3, uHyper, iHyper):
    n_total = x0.shape[0]
    bl = 1000
    nb = n_total // bl
    half_blocks = nb // 2

    def body(x0r, l1r, l2r, l3r, uhr, ihr, emb_r, hyp_r, gscr):
        i = pl.program_id(0)

        @pl.when(i == 0)
        def _():
            gscr[0] = lax.dot_general(uhr[...], uhr[...],
                                      (((0,), (0,)), ((), ())),
                                      preferred_element_type=jnp.float32)
            gscr[1] = lax.dot_general(ihr[...], ihr[...],
                                      (((0,), (0,)), ((), ())),
                                      preferred_element_type=jnp.float32)

        e = x0r[...] + l1r[...] + l2r[...] + l3r[...]
        emb_r[...] = e
        g = jnp.where(i < half_blocks, gscr[0], gscr[1])
        hyp_r[...] = jnp.dot(e, g, preferred_element_type=jnp.float32)

    blk = pl.BlockSpec((bl, D), lambda i: (i, 0))
    full = pl.BlockSpec(uHyper.shape, lambda i: (0, 0))
    return pl.pallas_call(
        body,
        grid=(nb,),
        in_specs=[blk, blk, blk, blk, full, full],
        out_specs=[blk, blk],
        out_shape=[jax.ShapeDtypeStruct((n_total, D), jnp.float32),
                   jax.ShapeDtypeStruct((n_total, D), jnp.float32)],
        scratch_shapes=[pltpu.VMEM((2, D, D), jnp.float32)],
    )(x0, l1, l2, l3, uHyper, iHyper)


def kernel(adj_indices, adj_values, uEmbeds, iEmbeds, uHyper, iHyper):
    n_user = uEmbeds.shape[0]
    e = adj_values.shape[0]

    val_bits = lax.bitcast_convert_type(
        adj_values.astype(jnp.bfloat16), jnp.uint16).astype(jnp.uint32)
    payload = (val_bits << 14) | adj_indices[1].astype(jnp.uint32)
    dst_s, pay_s = lax.sort((adj_indices[0], payload), num_keys=1)
    src_s = (pay_s & 0x3FFF).astype(jnp.int32)
    val_s = lax.bitcast_convert_type(
        (pay_s >> 14).astype(jnp.uint16), jnp.bfloat16).astype(jnp.float32)

    pad = 2 * QUAD
    e_cap = e + pad
    zpad_i = jnp.zeros((pad,), jnp.int32)
    src_p = jnp.concatenate([src_s, zpad_i])
    dstl_p = jnp.concatenate([dst_s, zpad_i])
    val_p = jnp.concatenate([val_s, jnp.zeros((pad,), jnp.float32)])

    bounds = jnp.arange(N_WORKERS + 1, dtype=jnp.int32) * RPT
    offs = jnp.sum(dst_s[:, None] < bounds[None, :], axis=0,
                   dtype=jnp.int32)
    start8 = (offs[:-1] // 8) * 8
    nch = -(-(offs[1:] - start8) // C)
    nch4 = (-(-nch // 4) * 4).astype(jnp.int32)
    meta = jnp.concatenate([
        jnp.broadcast_to(nch4[:, None], (N_WORKERS, LANES)).reshape(-1),
        jnp.broadcast_to(start8[:, None], (N_WORKERS, LANES)).reshape(-1),
    ]).astype(jnp.int32)

    x0 = jnp.concatenate([uEmbeds, iEmbeds], axis=0)
    lats = [x0]
    for _ in range(N_LAYERS):
        lats.append(_sc_spmm(lats[-1], src_p, dstl_p, val_p, meta, e_cap=e_cap))

    embeds, hyper = _tc_finish(lats[0], lats[1], lats[2], lats[3],
                               uHyper, iHyper)
    return (embeds, hyper[:n_user], hyper[n_user:])
